# baseline (speedup 1.0000x reference)
import functools

import jax
import jax.numpy as jnp
from jax import lax
from jax.experimental import pallas as pl
from jax.experimental.pallas import tpu as pltpu
from jax.experimental.pallas import tpu_sc as plsc

EMB = 64
_NC = 2
_NS = 16
_NW = _NC * _NS


@functools.lru_cache(maxsize=None)
def _make_gather(ids_shape, n_chunks: int, chunk: int):
    B = ids_shape[0] * ids_shape[1]
    b_per_w = B // _NW
    assert b_per_w == n_chunks * chunk

    mesh = plsc.VectorSubcoreMesh(core_axis_name="c", subcore_axis_name="s")

    @functools.partial(
        pl.kernel,
        mesh=mesh,
        out_type=jax.ShapeDtypeStruct((*ids_shape, EMB), jnp.float32),
        scratch_types=[
            pltpu.VMEM((chunk,), jnp.int32),
            pltpu.VMEM((chunk,), jnp.int32),
            pltpu.VMEM((chunk, EMB), jnp.float32),
            pltpu.VMEM((chunk, EMB), jnp.float32),
            pltpu.SemaphoreType.DMA,
            pltpu.SemaphoreType.DMA,
        ],
        compiler_params=pltpu.CompilerParams(use_tc_tiling_on_sc=False),
    )
    def k(ids_hbm, table_hbm, out_hbm, idx0, idx1, rows0, rows1, gsem, osem):
        wid = lax.axis_index("s") * _NC + lax.axis_index("c")
        seq, nb = ids_shape[1], chunk // ids_shape[1]
        idx_v = (idx0, idx1)
        rows_v = (rows0, rows1)

        def idx_src(j):
            return ids_hbm.at[pl.ds((wid * n_chunks + j) * chunk, chunk)]

        def start_out(j):
            b0 = (wid * n_chunks + j) * nb
            for i in range(nb):
                pltpu.async_copy(
                    rows_v[j % 2].at[pl.ds(i * seq, seq)],
                    out_hbm.at[b0 + i],
                    osem,
                )

        def wait_out(j):
            b0 = (wid * n_chunks + j) * nb
            for i in range(nb):
                pltpu.make_async_copy(
                    rows_v[j % 2].at[pl.ds(i * seq, seq)],
                    out_hbm.at[b0 + i],
                    osem,
                ).wait()

        pltpu.sync_copy(idx_src(0), idx0)
        pltpu.async_copy(table_hbm.at[idx0], rows0, gsem)
        for j in range(n_chunks):
            cur, nxt = j % 2, (j + 1) % 2
            if j + 1 < n_chunks:
                pltpu.sync_copy(idx_src(j + 1), idx_v[nxt])
                if j >= 1:
                    wait_out(j - 1)
                pltpu.async_copy(table_hbm.at[idx_v[nxt]], rows_v[nxt], gsem)
            pltpu.make_async_copy(
                table_hbm.at[idx_v[cur]], rows_v[cur], gsem
            ).wait()
            start_out(j)
        wait_out(n_chunks - 2)
        wait_out(n_chunks - 1)

    return k


def kernel(input_ids, table):
    ids_flat = input_ids.reshape(-1).astype(jnp.int32)
    return _make_gather(tuple(input_ids.shape), 8, 800)(ids_flat, table)
